# baseline (speedup 1.0000x reference)
import dataclasses
import functools

import jax
import jax.numpy as jnp
from jax import lax
from jax.experimental import pallas as pl
from jax.experimental.pallas import tpu as pltpu
from jax.experimental.pallas import tpu_sc as plsc

B = 16
TOTAL = 16384
D_FEAT = 256
D_MODEL = 1024
AUX_OUT = 1
TILE = 2048
NUM_TILES = TOTAL // TILE

L = 16
NW = 16
CHUNK = TOTAL // NW


def _tc_body(ent_ref, we_ref, be_ref, wb_ref, bb_ref, wa_ref, s_ref):
    x = jnp.dot(ent_ref[...], we_ref[...], preferred_element_type=jnp.float32)
    x = jnp.maximum(x + be_ref[...], 0.0)
    h = jnp.dot(x, wb_ref[...], preferred_element_type=jnp.float32)
    h = jnp.maximum(h + bb_ref[...], 0.0)
    s_ref[...] = jnp.dot(h, wa_ref[...], preferred_element_type=jnp.float32)


def _tc_stage(entities, W_embed, b_embed, W_bb, b_bb, W_aux):
    return pl.pallas_call(
        _tc_body,
        grid=(NUM_TILES,),
        in_specs=[
            pl.BlockSpec((TILE, D_FEAT), lambda i: (i, 0)),
            pl.BlockSpec((D_FEAT, D_MODEL), lambda i: (0, 0)),
            pl.BlockSpec((1, D_MODEL), lambda i: (0, 0)),
            pl.BlockSpec((D_MODEL, D_MODEL), lambda i: (0, 0)),
            pl.BlockSpec((1, D_MODEL), lambda i: (0, 0)),
            pl.BlockSpec((D_MODEL, AUX_OUT), lambda i: (0, 0)),
        ],
        out_specs=pl.BlockSpec((TILE, AUX_OUT), lambda i: (i, 0)),
        out_shape=jax.ShapeDtypeStruct((TOTAL, AUX_OUT), jnp.float32),
    )(entities, W_embed, b_embed.reshape(1, D_MODEL), W_bb,
      b_bb.reshape(1, D_MODEL), W_aux)


def _sc_pool(s_flat, bi32, baux_vec):
    mesh = plsc.VectorSubcoreMesh(core_axis_name="c", subcore_axis_name="s",
                                  num_cores=2, num_subcores=16)
    cp = pltpu.CompilerParams()
    if "needs_layout_passes" in pltpu.CompilerParams.__dataclass_fields__:
        cp = dataclasses.replace(cp, needs_layout_passes=False)

    @functools.partial(
        pl.kernel,
        compiler_params=cp,
        out_type=jax.ShapeDtypeStruct((B,), jnp.float32),
        mesh=mesh,
        scratch_types=[
            pltpu.VMEM((CHUNK,), jnp.float32),
            pltpu.VMEM((CHUNK,), jnp.int32),
            pltpu.VMEM((B * L,), jnp.float32),
            pltpu.VMEM((B * L,), jnp.float32),
            pltpu.VMEM((B,), jnp.float32),
            pltpu.VMEM((B,), jnp.float32),
            pltpu.VMEM((B,), jnp.float32),
            pltpu.VMEM((B,), jnp.float32),
            pltpu.VMEM_SHARED((B,), jnp.float32),
            pltpu.VMEM_SHARED((B,), jnp.float32),
        ],
    )
    def _sc_kernel(s_hbm, bi_hbm, baux_hbm, out_hbm,
                   s_v, bi_v, acc_v, cnt_v, tsum_v, tcnt_v, bax_v, out_v,
                   sh_sum, sh_cnt):
        core = lax.axis_index("c")
        sid = lax.axis_index("s")

        @pl.when(core == 0)
        def _():
            base = sid * CHUNK
            pltpu.sync_copy(s_hbm.at[pl.ds(base, CHUNK)], s_v)
            pltpu.sync_copy(bi_hbm.at[pl.ds(base, CHUNK)], bi_v)

            zeros = jnp.zeros((L,), jnp.float32)
            ones = jnp.ones((L,), jnp.float32)
            lane = lax.iota(jnp.int32, L)

            @pl.loop(0, B * L, step=L)
            def _zero(o):
                acc_v[pl.ds(o, L)] = zeros
                cnt_v[pl.ds(o, L)] = zeros

            @pl.loop(0, CHUNK, step=L)
            def _accum(o):
                sv = s_v[pl.ds(o, L)]
                bv = bi_v[pl.ds(o, L)]
                idx = bv * L + lane
                plsc.addupdate_scatter(acc_v, [idx], sv)
                plsc.addupdate_scatter(cnt_v, [idx], ones)

            colbase = lane * L
            sums = zeros
            cnts = zeros
            for k in range(L):
                sums = sums + plsc.load_gather(acc_v, [colbase + k])
                cnts = cnts + plsc.load_gather(cnt_v, [colbase + k])

            @pl.when(sid == 0)
            def _zero_shared():
                out_v[...] = jnp.zeros((L,), jnp.float32)
                pltpu.sync_copy(out_v, sh_sum)
                pltpu.sync_copy(out_v, sh_cnt)

            plsc.subcore_barrier()
            tsum_v[...] = sums
            tcnt_v[...] = cnts
            pltpu.sync_copy(tsum_v, sh_sum.at[lane], add=True)
            pltpu.sync_copy(tcnt_v, sh_cnt.at[lane], add=True)
            plsc.subcore_barrier()

            @pl.when(sid == 0)
            def _final():
                pltpu.sync_copy(sh_sum, tsum_v)
                pltpu.sync_copy(sh_cnt, tcnt_v)
                pltpu.sync_copy(baux_hbm, bax_v)
                res = (tsum_v[...] / jnp.maximum(tcnt_v[...], 1.0)
                       + bax_v[...])
                out_v[...] = res
                pltpu.sync_copy(out_v, out_hbm)

    return _sc_kernel(s_flat, bi32, baux_vec)


@jax.jit
def kernel(entities, batch_index, W_embed, b_embed, W_bb, b_bb, W_aux, b_aux):
    bi32 = batch_index.astype(jnp.int32)
    s = _tc_stage(entities, W_embed, b_embed, W_bb, b_bb, W_aux)
    baux_vec = jnp.broadcast_to(b_aux.astype(jnp.float32), (B,))
    out16 = _sc_pool(s.reshape(TOTAL), bi32, baux_vec)
    return out16.reshape(B, AUX_OUT)


# speedup vs baseline: 1.5102x; 1.5102x over previous
import functools

import jax
import jax.numpy as jnp
from jax.experimental import pallas as pl
from jax.experimental.pallas import tpu as pltpu

B = 16
TOTAL = 16384
D_FEAT = 256
D_MODEL = 1024
AUX_OUT = 1
TILE = 2048
NUM_TILES = TOTAL // TILE


def _fused_kernel(ent_ref, bi_ref, we_ref, be_ref, wb_ref, bb_ref, wa_ref,
                  ba_ref, out_ref, acc_ref, cnt_ref):
    i = pl.program_id(0)

    @pl.when(i == 0)
    def _init():
        acc_ref[...] = jnp.zeros_like(acc_ref)
        cnt_ref[...] = jnp.zeros_like(cnt_ref)

    x = jnp.dot(ent_ref[...], we_ref[...], preferred_element_type=jnp.float32)
    x = jnp.maximum(x + be_ref[...], 0.0)
    h = jnp.dot(x, wb_ref[...], preferred_element_type=jnp.float32)
    h = jnp.maximum(h + bb_ref[...], 0.0)

    bi = bi_ref[0, :]
    oh_t = (jax.lax.broadcasted_iota(jnp.int32, (B, TILE), 0)
            == bi[None, :]).astype(jnp.float32)
    acc_ref[...] += jnp.dot(oh_t, h, preferred_element_type=jnp.float32)
    cnt_ref[...] += jnp.broadcast_to(
        jnp.sum(oh_t, axis=1, keepdims=True), (B, 128))

    @pl.when(i == NUM_TILES - 1)
    def _finalize():
        counts = cnt_ref[:, 0:1]
        pooled = acc_ref[...] / jnp.maximum(counts, 1.0)
        aux = jnp.dot(pooled, wa_ref[...], preferred_element_type=jnp.float32)
        out_ref[...] = aux + ba_ref[...]


@functools.partial(jax.jit, static_argnames=())
def kernel(entities, batch_index, W_embed, b_embed, W_bb, b_bb, W_aux, b_aux):
    bi = batch_index.astype(jnp.int32).reshape(NUM_TILES, 1, TILE)
    grid = (NUM_TILES,)
    out = pl.pallas_call(
        _fused_kernel,
        grid=grid,
        in_specs=[
            pl.BlockSpec((TILE, D_FEAT), lambda i: (i, 0)),
            pl.BlockSpec((None, 1, TILE), lambda i: (i, 0, 0)),
            pl.BlockSpec((D_FEAT, D_MODEL), lambda i: (0, 0)),
            pl.BlockSpec((1, D_MODEL), lambda i: (0, 0)),
            pl.BlockSpec((D_MODEL, D_MODEL), lambda i: (0, 0)),
            pl.BlockSpec((1, D_MODEL), lambda i: (0, 0)),
            pl.BlockSpec((D_MODEL, AUX_OUT), lambda i: (0, 0)),
            pl.BlockSpec((1, AUX_OUT), lambda i: (0, 0)),
        ],
        out_specs=pl.BlockSpec((B, AUX_OUT), lambda i: (0, 0)),
        out_shape=jax.ShapeDtypeStruct((B, AUX_OUT), jnp.float32),
        scratch_shapes=[
            pltpu.VMEM((B, D_MODEL), jnp.float32),
            pltpu.VMEM((B, 128), jnp.float32),
        ],
    )(entities, bi, W_embed, b_embed.reshape(1, D_MODEL), W_bb,
      b_bb.reshape(1, D_MODEL), W_aux, b_aux.reshape(1, AUX_OUT))
    return out
